# Initial kernel scaffold; baseline (speedup 1.0000x reference)
import jax
import jax.numpy as jnp
from jax.experimental import pallas as pl


def kernel(psm, rm, anchor_box, transformation_matrix, target):
    raise NotImplementedError("write your pallas kernel here")


# speedup vs baseline: 26.0669x; 26.0669x over previous
import jax
import jax.numpy as jnp
from jax.experimental import pallas as pl
from jax.experimental.pallas import tpu as pltpu

W, L, NA, NT = 100, 100, 2, 50


def _loss_body(psm_ref, rm_ref, anc_ref, t_ref, tgt_ref, out_ref):
    def sc(i, j):
        return t_ref[i:i + 1, j:j + 1]

    r00, r01, r02, t0 = sc(0, 0), sc(0, 1), sc(0, 2), sc(0, 3)
    r10, r11, r12, t1 = sc(1, 0), sc(1, 1), sc(1, 2), sc(1, 3)

    tx = tgt_ref[0:1, :]
    ty = tgt_ref[1:2, :]
    th = tgt_ref[3:4, :]
    tw = tgt_ref[4:5, :]
    tl = tgt_ref[5:6, :]
    try_ = tgt_ref[6:7, :]
    tc, ts = jnp.cos(try_), jnp.sin(try_)
    tex = jnp.abs(tc) * tl * 0.5 + jnp.abs(ts) * tw * 0.5
    tey = jnp.abs(ts) * tl * 0.5 + jnp.abs(tc) * tw * 0.5
    gxmin, gxmax = tx - tex, tx + tex
    gymin, gymax = ty - tey, ty + tey
    garea = (gxmax - gxmin) * (gymax - gymin)
    del th

    total = jnp.float32(0.0)
    for a in range(NA):
        lg = psm_ref[a]
        prob = jax.nn.sigmoid(lg)
        wgt = jnp.where(prob > 0.1, jnp.log(1.0 - prob), 0.0)

        d = [rm_ref[a * 7 + c] for c in range(7)]
        an = [anc_ref[a * 7 + c] for c in range(7)]
        ad = jnp.sqrt(an[4] * an[4] + an[5] * an[5])
        bx = d[0] * ad + an[0]
        by = d[1] * ad + an[1]
        bz = d[2] * an[3] + an[2]
        dh = jnp.exp(d[3]) * an[3]
        dw = jnp.exp(d[4]) * an[4]
        dl = jnp.exp(d[5]) * an[5]
        ry = d[6] + an[6]
        c, s = jnp.cos(ry), jnp.sin(ry)

        cx = r00 * bx + r01 * by + r02 * bz + t0
        cy = r10 * bx + r11 * by + r12 * bz + t1
        ex = (jnp.abs(dl * 0.5 * (r00 * c + r01 * s))
              + jnp.abs(dw * 0.5 * (r01 * c - r00 * s))
              + jnp.abs(dh * 0.5 * r02))
        ey = (jnp.abs(dl * 0.5 * (r10 * c + r11 * s))
              + jnp.abs(dw * 0.5 * (r11 * c - r10 * s))
              + jnp.abs(dh * 0.5 * r12))
        pxmin, pxmax = cx - ex, cx + ex
        pymin, pymax = cy - ey, cy + ey
        parea = (pxmax - pxmin) * (pymax - pymin)

        iousum = jnp.zeros((W, L), jnp.float32)
        for n in range(NT):
            xm = gxmin[0:1, n:n + 1]
            xM = gxmax[0:1, n:n + 1]
            ym = gymin[0:1, n:n + 1]
            yM = gymax[0:1, n:n + 1]
            ga = garea[0:1, n:n + 1]
            iw = jnp.maximum(0.0, jnp.minimum(pxmax, xM) - jnp.maximum(pxmin, xm))
            ih = jnp.maximum(0.0, jnp.minimum(pymax, yM) - jnp.maximum(pymin, ym))
            inter = iw * ih
            iousum = iousum + inter / (parea + ga - inter)
        total = total + jnp.sum(wgt * iousum)

    out_ref[0, 0] = total


def kernel(psm, rm, anchor_box, transformation_matrix, target):
    psm3 = psm[0]
    rm3 = rm[0]
    anc = jnp.transpose(anchor_box, (2, 3, 0, 1)).reshape(7 * NA, W, L)
    tgt = jnp.transpose(target)
    out = pl.pallas_call(
        _loss_body,
        out_shape=jax.ShapeDtypeStruct((1, 1), jnp.float32),
        out_specs=pl.BlockSpec(memory_space=pltpu.SMEM),
    )(psm3, rm3, anc, transformation_matrix.astype(jnp.float32), tgt)
    return out[0, 0]
